# baseline (speedup 1.0000x reference)
import functools
import math

import jax
import jax.numpy as jnp
from jax.experimental import pallas as pl
from jax.experimental.pallas import tpu as pltpu

_DEPTH = 4
_RATIO = 0.5


def _pad_up(n, m=256):
    return ((n + m - 1) // m) * m


def _pick_block(dim, cap, mult):
    b = mult
    for c in range(mult, cap + 1, mult):
        if dim % c == 0:
            b = c
    return b


def _mm_body(a_ref, b_ref, o_ref, acc_ref, *, nk, trans_b, diag_one, relu, bm, bn):
    @pl.when(pl.program_id(2) == 0)
    def _init():
        acc_ref[...] = jnp.zeros_like(acc_ref)

    a = a_ref[...]
    b = b_ref[...]
    if a.dtype != b.dtype:
        a = a.astype(jnp.float32)
        b = b.astype(jnp.float32)
    if trans_b:
        acc_ref[...] += jax.lax.dot_general(
            a, b, (((1,), (1,)), ((), ())), preferred_element_type=jnp.float32)
    else:
        acc_ref[...] += jax.lax.dot_general(
            a, b, (((1,), (0,)), ((), ())), preferred_element_type=jnp.float32)

    @pl.when(pl.program_id(2) == nk - 1)
    def _fin():
        acc = acc_ref[...]
        if diag_one is not None:
            row = jax.lax.broadcasted_iota(jnp.int32, (bm, bn), 0) + pl.program_id(0) * bm
            col = jax.lax.broadcasted_iota(jnp.int32, (bm, bn), 1) + pl.program_id(1) * bn
            acc = jnp.where((row < diag_one) & (col < diag_one), acc, 0.0)
            acc = jnp.where(row == col, 1.0, acc)
        if relu:
            acc = jnp.maximum(acc, 0.0)
        o_ref[...] = acc.astype(o_ref.dtype)


def _matmul(a, b, *, trans_b=False, diag_one=None, relu=False, out_dtype=jnp.float32):
    m, k = a.shape
    if trans_b:
        n, kb = b.shape
    else:
        kb, n = b.shape
    bm = _pick_block(m, 1024, 8)
    bn = _pick_block(n, 1024, 128)
    bk = _pick_block(k, 512, 128)
    assert k == kb and m % bm == 0 and n % bn == 0 and k % bk == 0, (a.shape, b.shape)
    nk = k // bk
    if trans_b:
        b_spec = pl.BlockSpec((bn, bk), lambda i, j, s: (j, s))
    else:
        b_spec = pl.BlockSpec((bk, bn), lambda i, j, s: (s, j))
    return pl.pallas_call(
        functools.partial(_mm_body, nk=nk, trans_b=trans_b, diag_one=diag_one,
                          relu=relu, bm=bm, bn=bn),
        grid=(m // bm, n // bn, nk),
        in_specs=[pl.BlockSpec((bm, bk), lambda i, j, s: (i, s)), b_spec],
        out_specs=pl.BlockSpec((bm, bn), lambda i, j, s: (i, j)),
        out_shape=jax.ShapeDtypeStruct((m, n), out_dtype),
        scratch_shapes=[pltpu.VMEM((bm, bn), jnp.float32)],
        compiler_params=pltpu.CompilerParams(
            dimension_semantics=("parallel", "parallel", "arbitrary")),
    )(a, b)


def _gcn_B(SB, e, dis, fix, Y, b, relu):
    Yn = dis[:, None] * Y
    Z = _matmul(SB, Yn)
    out = dis[:, None] * (Z - e[:, None] * Yn) + (dis * dis * fix)[:, None] * Y + b[None, :]
    if relu:
        out = jnp.maximum(out, 0.0)
    return out


def _norm_B(SB, e, dA, n):
    p = SB.shape[0]
    real = jnp.arange(p) < n
    fix = jnp.where((dA == 0.0) & real, 2.0, 0.0)
    deg = jnp.sum(SB.astype(jnp.float32), axis=1) - e + fix
    dis = jnp.where(deg > 0, 1.0 / jnp.sqrt(jnp.maximum(deg, 1e-30)), 0.0)
    return dis, fix


def kernel(x, edge_index, W_down, b_down, pool_w, W_up, b_up):
    n0 = x.shape[0]
    p0 = _pad_up(n0)
    hid = W_down[0].shape[1]
    f32 = jnp.float32

    e0 = edge_index[0]
    e1 = edge_index[1]
    use_bf16 = n0 >= 4096
    adt = jnp.bfloat16 if use_bf16 else f32
    selfe = e0 == e1
    ones = jnp.where(selfe, 0.0, 1.0).astype(adt)
    ii = jnp.arange(n0, dtype=jnp.int32)
    eye1 = jnp.ones((n0,), adt)
    B = jnp.zeros((p0, p0), adt).at[e0, e1].add(ones).at[ii, ii].add(eye1)
    Bt = jnp.zeros((p0, p0), adt).at[e1, e0].add(ones).at[ii, ii].add(eye1)
    dA = jnp.zeros((p0,), f32).at[e0].add(jnp.where(selfe, 1.0, 0.0))

    e0v = jnp.where(jnp.arange(p0) < n0, 1.0 - dA, 0.0)
    dis, fix = _norm_B(B, e0v, dA, n0)
    kdim = _pad_up(x.shape[1])
    x_p = jnp.pad(x, ((0, p0 - n0), (0, kdim - x.shape[1])))
    W0_p = jnp.pad(W_down[0], ((0, kdim - x.shape[1]), (0, 0)))
    Y = _matmul(x_p, W0_p)
    h = _gcn_B(B, e0v, dis, fix, Y, b_down[0], relu=True)

    xs = [h]
    adjs = [B]
    adj_e = [e0v]
    adj_dis = [dis]
    adj_fix = [fix]
    perms = []

    SB, SBt = B, Bt
    n = n0
    for i in range(1, _DEPTH + 1):
        w = pool_w[i - 1]
        score = jnp.tanh((h[:n] @ w) / jnp.linalg.norm(w))
        k = int(math.ceil(_RATIO * n))
        pk = _pad_up(k)
        sv, perm = jax.lax.top_k(score, k)

        pn = SB.shape[0]
        perm_pad = jnp.concatenate(
            [perm, jnp.full((pk - k,), n, jnp.int32)]) if pk > k else perm
        G = SB[perm_pad]
        HT = SBt[perm_pad]
        s_dtype = adt if i <= 2 else f32
        S = _matmul(G, HT, trans_b=True, diag_one=k, out_dtype=s_dtype)

        sv_pad = jnp.concatenate([sv, jnp.zeros((pk - k,), f32)]) if pk > k else sv
        hp = h[perm_pad] * sv_pad[:, None]
        ev = jnp.where(jnp.arange(pk) < k, 1.0, 0.0)
        dis, fix = _norm_B(S, ev, jnp.zeros((pk,), f32), k)
        Y = _matmul(hp, W_down[i])
        h = _gcn_B(S, ev, dis, fix, Y, b_down[i], relu=True)

        if i < _DEPTH:
            xs.append(h)
            adjs.append(S)
            adj_e.append(ev)
            adj_dis.append(dis)
            adj_fix.append(fix)
        perms.append(perm)
        SB, SBt = S, S.T
        n = k

    for i in range(_DEPTH):
        j = _DEPTH - 1 - i
        res = xs[j]
        kj = perms[j].shape[0]
        W = W_up[i]
        out_p = _pad_up(W.shape[1], 128)
        Wa = jnp.pad(W[:hid], ((0, 0), (0, out_p - W.shape[1])))
        Wb = jnp.pad(W[hid:], ((0, 0), (0, out_p - W.shape[1])))
        b_p = jnp.pad(b_up[i], (0, out_p - b_up[i].shape[0]))
        hb = _matmul(h[:_pad_up(kj)], Wb)
        U = _matmul(res, Wa).at[perms[j]].add(hb[:kj])
        h = _gcn_B(adjs[j], adj_e[j], adj_dis[j], adj_fix[j], U, b_p,
                   relu=(i < _DEPTH - 1))

    return h[:n0, :W_up[-1].shape[1]]


# speedup vs baseline: 1.9716x; 1.9716x over previous
import functools
import math

import jax
import jax.numpy as jnp
from jax.experimental import pallas as pl
from jax.experimental.pallas import tpu as pltpu

_DEPTH = 4
_RATIO = 0.5


def _pad_up(n, m=256):
    return ((n + m - 1) // m) * m


def _pick_block(dim, cap, mult):
    b = mult
    for c in range(mult, cap + 1, mult):
        if dim % c == 0:
            b = c
    return b


def _mm_body(a_ref, b_ref, o_ref, acc_ref, *, nk, trans_b, diag_one, relu, bm, bn):
    @pl.when(pl.program_id(2) == 0)
    def _init():
        acc_ref[...] = jnp.zeros_like(acc_ref)

    a = a_ref[...]
    b = b_ref[...]
    if a.dtype != b.dtype:
        a = a.astype(jnp.float32)
        b = b.astype(jnp.float32)
    if trans_b:
        acc_ref[...] += jax.lax.dot_general(
            a, b, (((1,), (1,)), ((), ())), preferred_element_type=jnp.float32)
    else:
        acc_ref[...] += jax.lax.dot_general(
            a, b, (((1,), (0,)), ((), ())), preferred_element_type=jnp.float32)

    @pl.when(pl.program_id(2) == nk - 1)
    def _fin():
        acc = acc_ref[...]
        if diag_one is not None:
            row = jax.lax.broadcasted_iota(jnp.int32, (bm, bn), 0) + pl.program_id(0) * bm
            col = jax.lax.broadcasted_iota(jnp.int32, (bm, bn), 1) + pl.program_id(1) * bn
            acc = jnp.where((row < diag_one) & (col < diag_one), acc, 0.0)
            acc = jnp.where(row == col, 1.0, acc)
        if relu:
            acc = jnp.maximum(acc, 0.0)
        o_ref[...] = acc.astype(o_ref.dtype)


def _matmul(a, b, *, trans_b=False, diag_one=None, relu=False, out_dtype=jnp.float32):
    m, k = a.shape
    if trans_b:
        n, kb = b.shape
    else:
        kb, n = b.shape
    bm = _pick_block(m, 1280, 8)
    bn = _pick_block(n, 1280, 128)
    bk = _pick_block(k, 512, 128)
    assert k == kb and m % bm == 0 and n % bn == 0 and k % bk == 0, (a.shape, b.shape)
    nk = k // bk
    if trans_b:
        b_spec = pl.BlockSpec((bn, bk), lambda i, j, s: (j, s))
    else:
        b_spec = pl.BlockSpec((bk, bn), lambda i, j, s: (s, j))
    return pl.pallas_call(
        functools.partial(_mm_body, nk=nk, trans_b=trans_b, diag_one=diag_one,
                          relu=relu, bm=bm, bn=bn),
        grid=(m // bm, n // bn, nk),
        in_specs=[pl.BlockSpec((bm, bk), lambda i, j, s: (i, s)), b_spec],
        out_specs=pl.BlockSpec((bm, bn), lambda i, j, s: (i, j)),
        out_shape=jax.ShapeDtypeStruct((m, n), out_dtype),
        scratch_shapes=[pltpu.VMEM((bm, bn), jnp.float32)],
        compiler_params=pltpu.CompilerParams(
            dimension_semantics=("parallel", "parallel", "arbitrary")),
    )(a, b)


def _gcn_B(SB, e, dis, fix, Y, b, relu):
    Yn = dis[:, None] * Y
    Z = _matmul(SB, Yn)
    out = dis[:, None] * (Z - e[:, None] * Yn) + (dis * dis * fix)[:, None] * Y + b[None, :]
    if relu:
        out = jnp.maximum(out, 0.0)
    return out


def _norm_B(SB, e, dA, n):
    p = SB.shape[0]
    real = jnp.arange(p) < n
    fix = jnp.where((dA == 0.0) & real, 2.0, 0.0)
    deg = jnp.sum(SB.astype(jnp.float32), axis=1) - e + fix
    dis = jnp.where(deg > 0, 1.0 / jnp.sqrt(jnp.maximum(deg, 1e-30)), 0.0)
    return dis, fix


def kernel(x, edge_index, W_down, b_down, pool_w, W_up, b_up):
    n0 = x.shape[0]
    p0 = _pad_up(n0)
    hid = W_down[0].shape[1]
    f32 = jnp.float32

    e0 = edge_index[0]
    e1 = edge_index[1]
    use_bf16 = n0 >= 4096
    selfe = e0 == e1
    ones = jnp.where(selfe, 0.0, 1.0)
    ii = jnp.arange(n0, dtype=jnp.int32)
    B = jnp.zeros((p0, p0), f32).at[e0, e1].add(ones).at[ii, ii].add(1.0)
    Bt = jnp.zeros((p0, p0), f32).at[e1, e0].add(ones).at[ii, ii].add(1.0)
    dA = jnp.zeros((p0,), f32).at[e0].add(jnp.where(selfe, 1.0, 0.0))

    e0v = jnp.where(jnp.arange(p0) < n0, 1.0 - dA, 0.0)
    dis, fix = _norm_B(B, e0v, dA, n0)
    kdim = _pad_up(x.shape[1])
    x_p = jnp.pad(x, ((0, p0 - n0), (0, kdim - x.shape[1])))
    W0_p = jnp.pad(W_down[0], ((0, kdim - x.shape[1]), (0, 0)))
    Y = _matmul(x_p, W0_p)
    h = _gcn_B(B, e0v, dis, fix, Y, b_down[0], relu=True)

    xs = [h]
    adjs = [B]
    adj_e = [e0v]
    adj_dis = [dis]
    adj_fix = [fix]
    perms = []

    SB, SBt = B, Bt
    n = n0
    for i in range(1, _DEPTH + 1):
        w = pool_w[i - 1]
        score = jnp.tanh((h[:n] @ w) / jnp.linalg.norm(w))
        k = int(math.ceil(_RATIO * n))
        pk = _pad_up(k)
        sv, perm = jax.lax.top_k(score, k)

        pn = SB.shape[0]
        perm_pad = jnp.concatenate(
            [perm, jnp.full((pk - k,), n, jnp.int32)]) if pk > k else perm
        G = SB[perm_pad]
        HT = SBt[perm_pad]
        if use_bf16 and i <= 3:
            G = G.astype(jnp.bfloat16)
            HT = HT.astype(jnp.bfloat16)
        S = _matmul(G, HT, trans_b=True, diag_one=k)

        sv_pad = jnp.concatenate([sv, jnp.zeros((pk - k,), f32)]) if pk > k else sv
        hp = h[perm_pad] * sv_pad[:, None]
        ev = jnp.where(jnp.arange(pk) < k, 1.0, 0.0)
        dis, fix = _norm_B(S, ev, jnp.zeros((pk,), f32), k)
        Y = _matmul(hp, W_down[i])
        h = _gcn_B(S, ev, dis, fix, Y, b_down[i], relu=True)

        if i < _DEPTH:
            xs.append(h)
            adjs.append(S)
            adj_e.append(ev)
            adj_dis.append(dis)
            adj_fix.append(fix)
        perms.append(perm)
        SB, SBt = S, S.T
        n = k

    for i in range(_DEPTH):
        j = _DEPTH - 1 - i
        res = xs[j]
        kj = perms[j].shape[0]
        W = W_up[i]
        out_p = _pad_up(W.shape[1], 128)
        Wa = jnp.pad(W[:hid], ((0, 0), (0, out_p - W.shape[1])))
        Wb = jnp.pad(W[hid:], ((0, 0), (0, out_p - W.shape[1])))
        b_p = jnp.pad(b_up[i], (0, out_p - b_up[i].shape[0]))
        hb = _matmul(h[:_pad_up(kj)], Wb)
        U = _matmul(res, Wa).at[perms[j]].add(hb[:kj])
        h = _gcn_B(adjs[j], adj_e[j], adj_dis[j], adj_fix[j], U, b_p,
                   relu=(i < _DEPTH - 1))

    return h[:n0, :W_up[-1].shape[1]]
